# baseline (speedup 1.0000x reference)
import functools
import math

import jax
import jax.numpy as jnp
from jax.experimental import pallas as pl
from jax.experimental.pallas import tpu as pltpu

_BLKPT = 1.0
_WHTPT = 99.0
_MAX_MULT = 1.5

_ROWS = 128
_SROWS = 256
_NBITS = 30
_IMAX = 2**31 - 1


def _y_kernel(img_ref, m_ref, y_ref):
    r = img_ref[0, 0]
    g = img_ref[0, 1]
    b = img_ref[0, 2]
    y = r * m_ref[0, 0] + g * m_ref[0, 1] + b * m_ref[0, 2]
    y_ref[0] = jax.lax.bitcast_convert_type(y, jnp.int32)


def _select_kernel(y_ref, blk_ref, mul_ref, lo_ref, acc_ref, *,
                   nc, n, ranks, fracs):
    p = pl.program_id(1)
    c = pl.program_id(2)
    bits = y_ref[0]
    lanes = acc_ref.shape[-1]
    olanes = blk_ref.shape[-1]

    @pl.when((p == 0) & (c == 0))
    def _():
        lo_ref[0] = 0
        lo_ref[1] = 0

    @pl.when(c == 0)
    def _():
        acc_ref[0:2, :] = jnp.zeros((2, lanes), jnp.int32)
        acc_ref[2:4, :] = jnp.full((2, lanes), _IMAX, jnp.int32)

    @pl.when(p < _NBITS)
    def _():
        half = jnp.left_shift(jnp.int32(1), _NBITS - 1 - p)
        for r in range(2):
            t = lo_ref[r] + half
            acc_ref[r, :] += jnp.sum((bits >= t).astype(jnp.int32), axis=0)

        @pl.when(c == nc - 1)
        def _():
            for r in range(2):
                cge = jnp.sum(acc_ref[r, :])
                clt = n - cge
                lo_ref[r] = lo_ref[r] + jnp.where(
                    clt <= ranks[r], half, jnp.int32(0))

    @pl.when(p == _NBITS)
    def _():
        for r in range(2):
            v = lo_ref[r]
            le = bits <= v
            acc_ref[r, :] += jnp.sum(le.astype(jnp.int32), axis=0)
            above = jnp.where(le, _IMAX, bits)
            acc_ref[2 + r, :] = jnp.minimum(acc_ref[2 + r, :],
                                            jnp.min(above, axis=0))

        @pl.when(c == nc - 1)
        def _():
            vals = []
            for r in range(2):
                cnt_le = jnp.sum(acc_ref[r, :])
                mn_above = jnp.min(acc_ref[2 + r, :])
                lo_f = jax.lax.bitcast_convert_type(
                    jnp.full((olanes,), lo_ref[r], jnp.int32), jnp.float32)
                up_f = jax.lax.bitcast_convert_type(
                    jnp.full((olanes,), mn_above, jnp.int32), jnp.float32)
                take_lo = (jnp.full((olanes,), cnt_le, jnp.int32)
                           >= ranks[r] + 2)
                nxt = jnp.where(take_lo, lo_f, up_f)
                vals.append(lo_f * (1.0 - fracs[r]) + nxt * fracs[r])
            blkpt, whtpt = vals
            mult = jnp.minimum(1.0 / (whtpt - blkpt), _MAX_MULT)
            blk_ref[0, 0, :] = blkpt
            mul_ref[0, 0, :] = mult


def _apply_kernel(img_ref, blk_ref, mul_ref, out_ref):
    i = pl.program_id(0)
    bp = blk_ref[i]
    mu = mul_ref[i]
    out_ref[...] = jnp.clip((img_ref[...] - bp) * mu, 0.0, 1.0)


def kernel(image, rgb2yuv):
    B, C, H, W = image.shape
    n = H * W
    rows = _ROWS if H % _ROWS == 0 else H
    nc = H // rows
    srows = _SROWS if H % _SROWS == 0 else H
    snc = H // srows
    dims2 = ("parallel", "arbitrary")

    y_bits = pl.pallas_call(
        _y_kernel,
        grid=(B, nc),
        in_specs=[
            pl.BlockSpec((1, C, rows, W), lambda b, c: (b, 0, c, 0)),
            pl.BlockSpec(memory_space=pltpu.SMEM),
        ],
        out_specs=pl.BlockSpec((1, rows, W), lambda b, c: (b, c, 0)),
        out_shape=jax.ShapeDtypeStruct((B, H, W), jnp.int32),
        compiler_params=pltpu.CompilerParams(dimension_semantics=dims2),
    )(image, rgb2yuv)

    pos_b = _BLKPT / 100.0 * (n - 1)
    pos_w = _WHTPT / 100.0 * (n - 1)
    rank_b = int(math.floor(pos_b))
    rank_w = int(math.floor(pos_w))
    ranks = (rank_b, rank_w)
    fracs = (pos_b - rank_b, pos_w - rank_w)

    blk_v, mul_v = pl.pallas_call(
        functools.partial(_select_kernel, nc=snc, n=n, ranks=ranks,
                          fracs=fracs),
        grid=(B, _NBITS + 1, snc),
        in_specs=[pl.BlockSpec((1, srows, W), lambda b, p, c: (b, c, 0))],
        out_specs=[
            pl.BlockSpec((1, 1, 128), lambda b, p, c: (b, 0, 0)),
            pl.BlockSpec((1, 1, 128), lambda b, p, c: (b, 0, 0)),
        ],
        out_shape=[
            jax.ShapeDtypeStruct((B, 1, 128), jnp.float32),
            jax.ShapeDtypeStruct((B, 1, 128), jnp.float32),
        ],
        scratch_shapes=[
            pltpu.SMEM((2,), jnp.int32),
            pltpu.VMEM((4, W), jnp.int32),
        ],
        compiler_params=pltpu.CompilerParams(
            dimension_semantics=("parallel", "arbitrary", "arbitrary")),
    )(y_bits)

    blkpt = blk_v[:, 0, 0]
    mult = mul_v[:, 0, 0]

    return pl.pallas_call(
        _apply_kernel,
        grid=(B, nc),
        in_specs=[
            pl.BlockSpec((1, C, rows, W), lambda b, c: (b, 0, c, 0)),
            pl.BlockSpec(memory_space=pltpu.SMEM),
            pl.BlockSpec(memory_space=pltpu.SMEM),
        ],
        out_specs=pl.BlockSpec((1, C, rows, W), lambda b, c: (b, 0, c, 0)),
        out_shape=jax.ShapeDtypeStruct((B, C, H, W), jnp.float32),
        compiler_params=pltpu.CompilerParams(dimension_semantics=dims2),
    )(image, blkpt, mult)


# speedup vs baseline: 1.8952x; 1.8952x over previous
import functools
import math

import jax
import jax.numpy as jnp
from jax.experimental import pallas as pl
from jax.experimental.pallas import tpu as pltpu

_BLKPT = 1.0
_WHTPT = 99.0
_MAX_MULT = 1.5

_ROWS = 128
_NBITS = 30
_IMAX = 2**31 - 1


def _y_kernel(img_ref, m_ref, y_ref):
    r = img_ref[0, 0]
    g = img_ref[0, 1]
    b = img_ref[0, 2]
    y = r * m_ref[0, 0] + g * m_ref[0, 1] + b * m_ref[0, 2]
    y_ref[0] = jax.lax.bitcast_convert_type(y, jnp.int32)


def _select_kernel(y_ref, blk_ref, mul_ref, lo_ref, *, n, ranks, fracs):
    p = pl.program_id(1)
    bits = y_ref[0]
    olanes = blk_ref.shape[-1]

    @pl.when(p == 0)
    def _():
        lo_ref[0] = 0
        lo_ref[1] = 0

    @pl.when(p < _NBITS)
    def _():
        half = jnp.left_shift(jnp.int32(1), _NBITS - 1 - p)
        for r in range(2):
            t = lo_ref[r] + half
            cge = jnp.sum((bits >= t).astype(jnp.int32))
            clt = n - cge
            lo_ref[r] = lo_ref[r] + jnp.where(clt <= ranks[r], half,
                                              jnp.int32(0))

    @pl.when(p == _NBITS)
    def _():
        vals = []
        for r in range(2):
            v = lo_ref[r]
            le = bits <= v
            cnt_le = jnp.sum(le.astype(jnp.int32))
            mn_above = jnp.min(jnp.where(le, _IMAX, bits))
            lo_f = jax.lax.bitcast_convert_type(
                jnp.full((olanes,), v, jnp.int32), jnp.float32)
            up_f = jax.lax.bitcast_convert_type(
                jnp.full((olanes,), mn_above, jnp.int32), jnp.float32)
            take_lo = (jnp.full((olanes,), cnt_le, jnp.int32)
                       >= ranks[r] + 2)
            nxt = jnp.where(take_lo, lo_f, up_f)
            vals.append(lo_f * (1.0 - fracs[r]) + nxt * fracs[r])
        blkpt, whtpt = vals
        mult = jnp.minimum(1.0 / (whtpt - blkpt), _MAX_MULT)
        blk_ref[0, 0, :] = blkpt
        mul_ref[0, 0, :] = mult


def _apply_kernel(img_ref, blk_ref, mul_ref, out_ref):
    i = pl.program_id(0)
    bp = blk_ref[i]
    mu = mul_ref[i]
    out_ref[...] = jnp.clip((img_ref[...] - bp) * mu, 0.0, 1.0)


def kernel(image, rgb2yuv):
    B, C, H, W = image.shape
    n = H * W
    rows = _ROWS if H % _ROWS == 0 else H
    nc = H // rows
    dims2 = ("parallel", "arbitrary")

    y_bits = pl.pallas_call(
        _y_kernel,
        grid=(B, nc),
        in_specs=[
            pl.BlockSpec((1, C, rows, W), lambda b, c: (b, 0, c, 0)),
            pl.BlockSpec(memory_space=pltpu.SMEM),
        ],
        out_specs=pl.BlockSpec((1, rows, W), lambda b, c: (b, c, 0)),
        out_shape=jax.ShapeDtypeStruct((B, H, W), jnp.int32),
        compiler_params=pltpu.CompilerParams(dimension_semantics=dims2),
    )(image, rgb2yuv)

    pos_b = _BLKPT / 100.0 * (n - 1)
    pos_w = _WHTPT / 100.0 * (n - 1)
    rank_b = int(math.floor(pos_b))
    rank_w = int(math.floor(pos_w))
    ranks = (rank_b, rank_w)
    fracs = (pos_b - rank_b, pos_w - rank_w)

    blk_v, mul_v = pl.pallas_call(
        functools.partial(_select_kernel, n=n, ranks=ranks, fracs=fracs),
        grid=(B, _NBITS + 1),
        in_specs=[pl.BlockSpec((1, H, W), lambda b, p: (b, 0, 0))],
        out_specs=[
            pl.BlockSpec((1, 1, 128), lambda b, p: (b, 0, 0)),
            pl.BlockSpec((1, 1, 128), lambda b, p: (b, 0, 0)),
        ],
        out_shape=[
            jax.ShapeDtypeStruct((B, 1, 128), jnp.float32),
            jax.ShapeDtypeStruct((B, 1, 128), jnp.float32),
        ],
        scratch_shapes=[
            pltpu.SMEM((2,), jnp.int32),
        ],
        compiler_params=pltpu.CompilerParams(
            dimension_semantics=("parallel", "arbitrary"),
            vmem_limit_bytes=48 * 1024 * 1024),
    )(y_bits)

    blkpt = blk_v[:, 0, 0]
    mult = mul_v[:, 0, 0]

    return pl.pallas_call(
        _apply_kernel,
        grid=(B, nc),
        in_specs=[
            pl.BlockSpec((1, C, rows, W), lambda b, c: (b, 0, c, 0)),
            pl.BlockSpec(memory_space=pltpu.SMEM),
            pl.BlockSpec(memory_space=pltpu.SMEM),
        ],
        out_specs=pl.BlockSpec((1, C, rows, W), lambda b, c: (b, 0, c, 0)),
        out_shape=jax.ShapeDtypeStruct((B, C, H, W), jnp.float32),
        compiler_params=pltpu.CompilerParams(dimension_semantics=dims2),
    )(image, blkpt, mult)


# speedup vs baseline: 2.4064x; 1.2698x over previous
import functools
import math

import jax
import jax.numpy as jnp
from jax.experimental import pallas as pl
from jax.experimental.pallas import tpu as pltpu

_BLKPT = 1.0
_WHTPT = 99.0
_MAX_MULT = 1.5

_ROWS = 128
_NBITS = 30
_IMAX = 2**31 - 1


def _y_kernel(img_ref, m_ref, y_ref, hi_ref):
    r = img_ref[0, 0]
    g = img_ref[0, 1]
    b = img_ref[0, 2]
    y = r * m_ref[0, 0] + g * m_ref[0, 1] + b * m_ref[0, 2]
    bits = jax.lax.bitcast_convert_type(y, jnp.int32)
    y_ref[0] = bits
    hi_ref[0] = (bits >> 15).astype(jnp.int16)


def _tree_count16(mask):
    x = mask.astype(jnp.int16)
    r = x.shape[0]
    while r > 8:
        h = r // 2
        x = x[:h] + x[h:]
        r = h
    return jnp.sum(x.astype(jnp.int32))


def _select_kernel(y_ref, hi_ref, blk_ref, mul_ref, lo_ref, *,
                   n, ranks, fracs):
    p = pl.program_id(1)
    olanes = blk_ref.shape[-1]

    @pl.when(p == 0)
    def _():
        lo_ref[0] = 0
        lo_ref[1] = 0

    @pl.when(p < _NBITS - 15)
    def _():
        hi = hi_ref[0]
        half = jnp.left_shift(jnp.int32(1), _NBITS - 1 - p)
        for r in range(2):
            t16 = ((lo_ref[r] + half) >> 15).astype(jnp.int16)
            cge = _tree_count16(hi >= t16)
            clt = n - cge
            lo_ref[r] = lo_ref[r] + jnp.where(clt <= ranks[r], half,
                                              jnp.int32(0))

    @pl.when((p >= _NBITS - 15) & (p < _NBITS))
    def _():
        bits = y_ref[0]
        half = jnp.left_shift(jnp.int32(1), _NBITS - 1 - p)
        for r in range(2):
            t = lo_ref[r] + half
            cge = jnp.sum((bits >= t).astype(jnp.int32))
            clt = n - cge
            lo_ref[r] = lo_ref[r] + jnp.where(clt <= ranks[r], half,
                                              jnp.int32(0))

    @pl.when(p == _NBITS)
    def _():
        bits = y_ref[0]
        vals = []
        for r in range(2):
            v = lo_ref[r]
            le = bits <= v
            cnt_le = jnp.sum(le.astype(jnp.int32))
            mn_above = jnp.min(jnp.where(le, _IMAX, bits))
            lo_f = jax.lax.bitcast_convert_type(
                jnp.full((olanes,), v, jnp.int32), jnp.float32)
            up_f = jax.lax.bitcast_convert_type(
                jnp.full((olanes,), mn_above, jnp.int32), jnp.float32)
            take_lo = (jnp.full((olanes,), cnt_le, jnp.int32)
                       >= ranks[r] + 2)
            nxt = jnp.where(take_lo, lo_f, up_f)
            vals.append(lo_f * (1.0 - fracs[r]) + nxt * fracs[r])
        blkpt, whtpt = vals
        mult = jnp.minimum(1.0 / (whtpt - blkpt), _MAX_MULT)
        blk_ref[0, 0, :] = blkpt
        mul_ref[0, 0, :] = mult


def _apply_kernel(img_ref, blk_ref, mul_ref, out_ref):
    i = pl.program_id(0)
    bp = blk_ref[i]
    mu = mul_ref[i]
    out_ref[...] = jnp.clip((img_ref[...] - bp) * mu, 0.0, 1.0)


def kernel(image, rgb2yuv):
    B, C, H, W = image.shape
    n = H * W
    rows = _ROWS if H % _ROWS == 0 else H
    nc = H // rows
    dims2 = ("parallel", "arbitrary")

    y_bits, y_hi = pl.pallas_call(
        _y_kernel,
        grid=(B, nc),
        in_specs=[
            pl.BlockSpec((1, C, rows, W), lambda b, c: (b, 0, c, 0)),
            pl.BlockSpec(memory_space=pltpu.SMEM),
        ],
        out_specs=[
            pl.BlockSpec((1, rows, W), lambda b, c: (b, c, 0)),
            pl.BlockSpec((1, rows, W), lambda b, c: (b, c, 0)),
        ],
        out_shape=[
            jax.ShapeDtypeStruct((B, H, W), jnp.int32),
            jax.ShapeDtypeStruct((B, H, W), jnp.int16),
        ],
        compiler_params=pltpu.CompilerParams(dimension_semantics=dims2),
    )(image, rgb2yuv)

    pos_b = _BLKPT / 100.0 * (n - 1)
    pos_w = _WHTPT / 100.0 * (n - 1)
    rank_b = int(math.floor(pos_b))
    rank_w = int(math.floor(pos_w))
    ranks = (rank_b, rank_w)
    fracs = (pos_b - rank_b, pos_w - rank_w)

    blk_v, mul_v = pl.pallas_call(
        functools.partial(_select_kernel, n=n, ranks=ranks, fracs=fracs),
        grid=(B, _NBITS + 1),
        in_specs=[
            pl.BlockSpec((1, H, W), lambda b, p: (b, 0, 0)),
            pl.BlockSpec((1, H, W), lambda b, p: (b, 0, 0)),
        ],
        out_specs=[
            pl.BlockSpec((1, 1, 128), lambda b, p: (b, 0, 0)),
            pl.BlockSpec((1, 1, 128), lambda b, p: (b, 0, 0)),
        ],
        out_shape=[
            jax.ShapeDtypeStruct((B, 1, 128), jnp.float32),
            jax.ShapeDtypeStruct((B, 1, 128), jnp.float32),
        ],
        scratch_shapes=[
            pltpu.SMEM((2,), jnp.int32),
        ],
        compiler_params=pltpu.CompilerParams(
            dimension_semantics=("parallel", "arbitrary"),
            vmem_limit_bytes=48 * 1024 * 1024),
    )(y_bits, y_hi)

    blkpt = blk_v[:, 0, 0]
    mult = mul_v[:, 0, 0]

    return pl.pallas_call(
        _apply_kernel,
        grid=(B, nc),
        in_specs=[
            pl.BlockSpec((1, C, rows, W), lambda b, c: (b, 0, c, 0)),
            pl.BlockSpec(memory_space=pltpu.SMEM),
            pl.BlockSpec(memory_space=pltpu.SMEM),
        ],
        out_specs=pl.BlockSpec((1, C, rows, W), lambda b, c: (b, 0, c, 0)),
        out_shape=jax.ShapeDtypeStruct((B, C, H, W), jnp.float32),
        compiler_params=pltpu.CompilerParams(dimension_semantics=dims2),
    )(image, blkpt, mult)


# speedup vs baseline: 2.8628x; 1.1896x over previous
import functools
import math

import jax
import jax.numpy as jnp
from jax.experimental import pallas as pl
from jax.experimental.pallas import tpu as pltpu

_BLKPT = 1.0
_WHTPT = 99.0
_MAX_MULT = 1.5

_ROWS = 128
_NBITS = 30
_NHI = 15
_IMAX = 2**31 - 1


def _y_kernel(img_ref, m_ref, hi_ref, lo_ref):
    r = img_ref[0, 0]
    g = img_ref[0, 1]
    b = img_ref[0, 2]
    y = r * m_ref[0, 0] + g * m_ref[0, 1] + b * m_ref[0, 2]
    bits = jax.lax.bitcast_convert_type(y, jnp.int32)
    hi_ref[0] = (bits >> _NHI).astype(jnp.int16)
    lo_ref[0] = (bits & 0x7FFF).astype(jnp.int16)


def _tree_count16(mask):
    x = mask.astype(jnp.int16)
    r = x.shape[0]
    while r > 8:
        h = r // 2
        x = x[:h] + x[h:]
        r = h
    return jnp.sum(x.astype(jnp.int32))


def _select_kernel(hi_ref, lo16_ref, blk_ref, mul_ref, st_ref, w_ref, *,
                   n, ranks, fracs):
    p = pl.program_id(1)
    olanes = blk_ref.shape[-1]

    @pl.when(p == 0)
    def _():
        st_ref[0] = 0
        st_ref[1] = 0

    @pl.when(p < _NBITS - _NHI)
    def _():
        hi = hi_ref[0]
        half = jnp.left_shift(jnp.int32(1), _NBITS - 1 - p)
        for r in range(2):
            t16 = ((st_ref[r] + half) >> _NHI).astype(jnp.int16)
            cge = _tree_count16(hi >= t16)
            clt = n - cge
            st_ref[r] = st_ref[r] + jnp.where(clt <= ranks[r], half,
                                              jnp.int32(0))

    @pl.when(p == _NBITS - _NHI)
    def _():
        hi = hi_ref[0]
        lo16 = lo16_ref[0]
        for r in range(2):
            h16 = (st_ref[r] >> _NHI).astype(jnp.int16)
            st_ref[2 + r] = _tree_count16(hi > h16)
            w_ref[r] = jnp.where(hi == h16, lo16, jnp.int16(-1))

    @pl.when((p >= _NBITS - _NHI) & (p < _NBITS))
    def _():
        half = jnp.left_shift(jnp.int32(1), _NBITS - 1 - p)
        for r in range(2):
            t = st_ref[r] + half
            t16 = (t & 0x7FFF).astype(jnp.int16)
            cge = st_ref[2 + r] + _tree_count16(w_ref[r] >= t16)
            clt = n - cge
            st_ref[r] = st_ref[r] + jnp.where(clt <= ranks[r], half,
                                              jnp.int32(0))

    @pl.when(p == _NBITS)
    def _():
        bits = (jnp.left_shift(hi_ref[0].astype(jnp.int32), _NHI)
                | lo16_ref[0].astype(jnp.int32))
        vals = []
        for r in range(2):
            v = st_ref[r]
            le = bits <= v
            cnt_le = jnp.sum(le.astype(jnp.int32))
            mn_above = jnp.min(jnp.where(le, _IMAX, bits))
            lo_f = jax.lax.bitcast_convert_type(
                jnp.full((olanes,), v, jnp.int32), jnp.float32)
            up_f = jax.lax.bitcast_convert_type(
                jnp.full((olanes,), mn_above, jnp.int32), jnp.float32)
            take_lo = (jnp.full((olanes,), cnt_le, jnp.int32)
                       >= ranks[r] + 2)
            nxt = jnp.where(take_lo, lo_f, up_f)
            vals.append(lo_f * (1.0 - fracs[r]) + nxt * fracs[r])
        blkpt, whtpt = vals
        mult = jnp.minimum(1.0 / (whtpt - blkpt), _MAX_MULT)
        blk_ref[0, 0, :] = blkpt
        mul_ref[0, 0, :] = mult


def _apply_kernel(img_ref, blk_ref, mul_ref, out_ref):
    i = pl.program_id(0)
    bp = blk_ref[i]
    mu = mul_ref[i]
    out_ref[...] = jnp.clip((img_ref[...] - bp) * mu, 0.0, 1.0)


def kernel(image, rgb2yuv):
    B, C, H, W = image.shape
    n = H * W
    rows = _ROWS if H % _ROWS == 0 else H
    nc = H // rows
    dims2 = ("parallel", "arbitrary")

    y_hi, y_lo = pl.pallas_call(
        _y_kernel,
        grid=(B, nc),
        in_specs=[
            pl.BlockSpec((1, C, rows, W), lambda b, c: (b, 0, c, 0)),
            pl.BlockSpec(memory_space=pltpu.SMEM),
        ],
        out_specs=[
            pl.BlockSpec((1, rows, W), lambda b, c: (b, c, 0)),
            pl.BlockSpec((1, rows, W), lambda b, c: (b, c, 0)),
        ],
        out_shape=[
            jax.ShapeDtypeStruct((B, H, W), jnp.int16),
            jax.ShapeDtypeStruct((B, H, W), jnp.int16),
        ],
        compiler_params=pltpu.CompilerParams(dimension_semantics=dims2),
    )(image, rgb2yuv)

    pos_b = _BLKPT / 100.0 * (n - 1)
    pos_w = _WHTPT / 100.0 * (n - 1)
    rank_b = int(math.floor(pos_b))
    rank_w = int(math.floor(pos_w))
    ranks = (rank_b, rank_w)
    fracs = (pos_b - rank_b, pos_w - rank_w)

    blk_v, mul_v = pl.pallas_call(
        functools.partial(_select_kernel, n=n, ranks=ranks, fracs=fracs),
        grid=(B, _NBITS + 1),
        in_specs=[
            pl.BlockSpec((1, H, W), lambda b, p: (b, 0, 0)),
            pl.BlockSpec((1, H, W), lambda b, p: (b, 0, 0)),
        ],
        out_specs=[
            pl.BlockSpec((1, 1, 128), lambda b, p: (b, 0, 0)),
            pl.BlockSpec((1, 1, 128), lambda b, p: (b, 0, 0)),
        ],
        out_shape=[
            jax.ShapeDtypeStruct((B, 1, 128), jnp.float32),
            jax.ShapeDtypeStruct((B, 1, 128), jnp.float32),
        ],
        scratch_shapes=[
            pltpu.SMEM((4,), jnp.int32),
            pltpu.VMEM((2, H, W), jnp.int16),
        ],
        compiler_params=pltpu.CompilerParams(
            dimension_semantics=("parallel", "arbitrary"),
            vmem_limit_bytes=48 * 1024 * 1024),
    )(y_hi, y_lo)

    blkpt = blk_v[:, 0, 0]
    mult = mul_v[:, 0, 0]

    return pl.pallas_call(
        _apply_kernel,
        grid=(B, nc),
        in_specs=[
            pl.BlockSpec((1, C, rows, W), lambda b, c: (b, 0, c, 0)),
            pl.BlockSpec(memory_space=pltpu.SMEM),
            pl.BlockSpec(memory_space=pltpu.SMEM),
        ],
        out_specs=pl.BlockSpec((1, C, rows, W), lambda b, c: (b, 0, c, 0)),
        out_shape=jax.ShapeDtypeStruct((B, C, H, W), jnp.float32),
        compiler_params=pltpu.CompilerParams(dimension_semantics=dims2),
    )(image, blkpt, mult)
